# baseline (speedup 1.0000x reference)
import jax
import jax.numpy as jnp
from jax.experimental import pallas as pl
from jax.experimental.pallas import tpu as pltpu

N_NODES = 10000
NFEAT = 128
NHID = 32
N_CLUSTERS = 10
ALPHA = 0.2
CH = 200
NCH = N_NODES // CH
NBUF = 4


def _chunk_copy(adj_hbm, buf, sem, chunk, slot):
    return pltpu.make_async_copy(
        adj_hbm.at[pl.ds(chunk * CH, CH), :], buf.at[slot], sem.at[slot]
    )


def _gcdec_body(w_ref, b_ref, mu_ref, x_hbm, adj_hbm, out_ref, q_ref,
                buf, x_vmem, support, sem, xsem):
    x_copy = pltpu.make_async_copy(x_hbm, x_vmem, xsem)
    x_copy.start()
    for k in range(NBUF):
        _chunk_copy(adj_hbm, buf, sem, k, k).start()
    x_copy.wait()

    support[:] = jnp.dot(x_vmem[:], w_ref[:], preferred_element_type=jnp.float32)
    mu = mu_ref[:]
    mu_sq = jnp.sum(mu * mu, axis=1, keepdims=True).reshape(1, N_CLUSTERS)

    for i in range(NCH):
        slot = i % NBUF
        _chunk_copy(adj_hbm, buf, sem, i, slot).wait()
        out_blk = (
            jnp.dot(buf[slot], support[:], preferred_element_type=jnp.float32)
            + b_ref[:]
        )
        if i + NBUF < NCH:
            _chunk_copy(adj_hbm, buf, sem, i + NBUF, slot).start()

        out_ref[pl.ds(i * CH, CH), :] = out_blk
        cross = jax.lax.dot_general(
            out_blk, mu, (((1,), (1,)), ((), ())),
            preferred_element_type=jnp.float32,
        )
        d2 = (
            jnp.sum(out_blk * out_blk, axis=1, keepdims=True) + mu_sq
            - 2.0 * cross
        )
        q = 1.0 / (1.0 + d2 / ALPHA + 1e-08)
        q = q ** (ALPHA + 1.0) / 2.0
        q_ref[pl.ds(i * CH, CH), :] = q / jnp.sum(q, axis=1, keepdims=True)


def kernel(x, adj, W, b, mu):
    b2 = b.reshape(1, NHID)
    out, q = pl.pallas_call(
        _gcdec_body,
        in_specs=[
            pl.BlockSpec((NFEAT, NHID), lambda: (0, 0)),
            pl.BlockSpec((1, NHID), lambda: (0, 0)),
            pl.BlockSpec((N_CLUSTERS, NHID), lambda: (0, 0)),
            pl.BlockSpec(memory_space=pltpu.MemorySpace.HBM),
            pl.BlockSpec(memory_space=pltpu.MemorySpace.HBM),
        ],
        out_specs=[
            pl.BlockSpec((N_NODES, NHID), lambda: (0, 0)),
            pl.BlockSpec((N_NODES, N_CLUSTERS), lambda: (0, 0)),
        ],
        out_shape=[
            jax.ShapeDtypeStruct((N_NODES, NHID), jnp.float32),
            jax.ShapeDtypeStruct((N_NODES, N_CLUSTERS), jnp.float32),
        ],
        scratch_shapes=[
            pltpu.VMEM((NBUF, CH, N_NODES), jnp.float32),
            pltpu.VMEM((N_NODES, NFEAT), jnp.float32),
            pltpu.VMEM((N_NODES, NHID), jnp.float32),
            pltpu.SemaphoreType.DMA((NBUF,)),
            pltpu.SemaphoreType.DMA,
        ],
        compiler_params=pltpu.CompilerParams(
            vmem_limit_bytes=64 * 1024 * 1024,
        ),
    )(W, b2, mu, x, adj)
    return (out, q)
